# baseline (speedup 1.0000x reference)
import functools

import jax
import jax.numpy as jnp
from jax.experimental import pallas as pl
from jax.experimental.pallas import tpu as pltpu

_NUM_CHUNKS = 8


def _dma_copy(emb_ref, out_ref, sem, *, num_chunks, rows_per_chunk):
    for c in range(num_chunks):
        sl = pl.ds(c * rows_per_chunk, rows_per_chunk)
        pltpu.make_async_copy(emb_ref.at[sl], out_ref.at[sl], sem).start()
    for c in range(num_chunks):
        sl = pl.ds(c * rows_per_chunk, rows_per_chunk)
        pltpu.make_async_copy(emb_ref.at[sl], out_ref.at[sl], sem).wait()


def kernel(inputs, embeddings):
    seq_len = inputs.shape[1]
    emb_dim = embeddings.shape[1]
    table = embeddings[:seq_len, :]
    num_chunks = _NUM_CHUNKS if seq_len % _NUM_CHUNKS == 0 else 1
    out = pl.pallas_call(
        functools.partial(
            _dma_copy,
            num_chunks=num_chunks,
            rows_per_chunk=seq_len // num_chunks,
        ),
        in_specs=[pl.BlockSpec(memory_space=pl.ANY)],
        out_specs=pl.BlockSpec(memory_space=pl.ANY),
        out_shape=jax.ShapeDtypeStruct((seq_len, emb_dim), embeddings.dtype),
        scratch_shapes=[pltpu.SemaphoreType.DMA],
    )(table)
    return out[None]


# speedup vs baseline: 38.7274x; 38.7274x over previous
import jax
import jax.numpy as jnp
from jax.experimental import pallas as pl
from jax.experimental.pallas import tpu as pltpu

_BLOCK_ROWS = 512


def _copy_block(emb_ref, out_ref):
    out_ref[...] = emb_ref[...]


def kernel(inputs, embeddings):
    seq_len = inputs.shape[1]
    emb_dim = embeddings.shape[1]
    table = embeddings[:seq_len, :]
    blk = min(_BLOCK_ROWS, seq_len)
    grid = (seq_len // blk,)
    out = pl.pallas_call(
        _copy_block,
        grid=grid,
        in_specs=[pl.BlockSpec((blk, emb_dim), lambda i: (i, 0))],
        out_specs=pl.BlockSpec((blk, emb_dim), lambda i: (i, 0)),
        out_shape=jax.ShapeDtypeStruct((seq_len, emb_dim), embeddings.dtype),
        compiler_params=pltpu.CompilerParams(
            dimension_semantics=("parallel",),
        ),
    )(table)
    return out[None]


# speedup vs baseline: 42.0443x; 1.0856x over previous
import jax
import jax.numpy as jnp
from jax.experimental import pallas as pl
from jax.experimental.pallas import tpu as pltpu

_BLOCK_ROWS = 1024


def _copy_block(emb_ref, out_ref):
    out_ref[...] = emb_ref[...]


def kernel(inputs, embeddings):
    seq_len = inputs.shape[1]
    emb_dim = embeddings.shape[1]
    table = embeddings[:seq_len, :]
    blk = min(_BLOCK_ROWS, seq_len)
    grid = (seq_len // blk,)
    out = pl.pallas_call(
        _copy_block,
        grid=grid,
        in_specs=[pl.BlockSpec((blk, emb_dim), lambda i: (i, 0))],
        out_specs=pl.BlockSpec((blk, emb_dim), lambda i: (i, 0)),
        out_shape=jax.ShapeDtypeStruct((seq_len, emb_dim), embeddings.dtype),
        compiler_params=pltpu.CompilerParams(
            dimension_semantics=("parallel",),
        ),
    )(table)
    return out[None]


# speedup vs baseline: 47.2390x; 1.1236x over previous
import jax
import jax.numpy as jnp
from jax.experimental import pallas as pl
from jax.experimental.pallas import tpu as pltpu

_BLOCK_ROWS = 2048


def _copy_block(emb_ref, out_ref):
    out_ref[...] = emb_ref[...]


def kernel(inputs, embeddings):
    seq_len = inputs.shape[1]
    emb_dim = embeddings.shape[1]
    table = embeddings[:seq_len, :]
    blk = min(_BLOCK_ROWS, seq_len)
    grid = (seq_len // blk,)
    out = pl.pallas_call(
        _copy_block,
        grid=grid,
        in_specs=[pl.BlockSpec((blk, emb_dim), lambda i: (i, 0))],
        out_specs=pl.BlockSpec((blk, emb_dim), lambda i: (i, 0)),
        out_shape=jax.ShapeDtypeStruct((seq_len, emb_dim), embeddings.dtype),
        compiler_params=pltpu.CompilerParams(
            dimension_semantics=("parallel",),
        ),
    )(table)
    return out[None]
